# Initial kernel scaffold; baseline (speedup 1.0000x reference)
import functools

import jax
import jax.numpy as jnp
from jax import lax
from jax.experimental import pallas as pl
from jax.experimental.pallas import tpu as pltpu
from jax.experimental.pallas import tpu_sc as plsc

N = 10000
E = 320000
D = 128
FE = 16

NC = 2
NS = 16
NW = NC * NS

K = 128
EPT = -(-E // (NW * K)) * K
NCH = EPT // K
E_PAD = EPT * NW

N_PAD = 10240
STRIPE = N_PAD // NS
DUMP = N

_NEG = -1e30


def _escore_body(ea_ref, we1_ref, ate1_ref, we2_ref, ate2_ref, o1_ref, o2_ref):
    ea = ea_ref[...]
    wv1 = jnp.dot(we1_ref[...], ate1_ref[...], preferred_element_type=jnp.float32)
    wv2 = jnp.dot(we2_ref[...], ate2_ref[...], preferred_element_type=jnp.float32)
    o1_ref[...] = jnp.dot(ea, wv1, preferred_element_type=jnp.float32)
    o2_ref[...] = jnp.dot(ea, wv2, preferred_element_type=jnp.float32)


def _edge_scores(edge_attr, We1, att_e1, We2, att_e2):
    return pl.pallas_call(
        _escore_body,
        out_shape=[
            jax.ShapeDtypeStruct((E, 1), jnp.float32),
            jax.ShapeDtypeStruct((E, 1), jnp.float32),
        ],
    )(edge_attr, We1, att_e1.reshape(D, 1), We2, att_e2.reshape(D, 1))


def _proj_body(x_ref, w_ref, ats_ref, atd_ref, h_ref, asrc_ref, adst_ref):
    h = jnp.dot(x_ref[...], w_ref[...], preferred_element_type=jnp.float32)
    h_ref[...] = h
    asrc_ref[...] = jnp.dot(h, ats_ref[...], preferred_element_type=jnp.float32)
    adst_ref[...] = jnp.dot(h, atd_ref[...], preferred_element_type=jnp.float32)


def _project(x, W, att_s, att_d):
    return pl.pallas_call(
        _proj_body,
        out_shape=[
            jax.ShapeDtypeStruct((N, D), jnp.float32),
            jax.ShapeDtypeStruct((N, 1), jnp.float32),
            jax.ShapeDtypeStruct((N, 1), jnp.float32),
        ],
    )(x, W, att_s.reshape(D, 1), att_d.reshape(D, 1))


_MESH = plsc.VectorSubcoreMesh(core_axis_name="c", subcore_axis_name="s")


@functools.partial(
    pl.kernel,
    out_type=[
        jax.ShapeDtypeStruct((NC, N_PAD, D), jnp.float32),
        jax.ShapeDtypeStruct((NC, N_PAD), jnp.float32),
        jax.ShapeDtypeStruct((NC, N_PAD), jnp.float32),
        jax.ShapeDtypeStruct((NC, N_PAD), jnp.float32),
    ],
    mesh=_MESH,
    scratch_types=[
        pltpu.VMEM((NCH, K), jnp.int32),
        pltpu.VMEM((NCH, K), jnp.int32),
        pltpu.VMEM((NCH, K), jnp.float32),
        pltpu.VMEM((NCH, K), jnp.float32),
        pltpu.VMEM((N_PAD,), jnp.float32),
        pltpu.VMEM((N_PAD,), jnp.float32),
        pltpu.VMEM((2, K, D), jnp.float32),
        pltpu.VMEM((K,), jnp.float32),
        pltpu.SemaphoreType.DMA,
        pltpu.SemaphoreType.DMA,
        pltpu.VMEM_SHARED((N_PAD, D), jnp.float32),
        pltpu.VMEM_SHARED((N_PAD,), jnp.float32),
        pltpu.VMEM_SHARED((N_PAD,), jnp.float32),
        pltpu.VMEM_SHARED((N_PAD,), jnp.float32),
    ],
)
def _sc_edge_pass(h_hbm, src_hbm, dst_hbm, ae_hbm, asrc_hbm, adst_hbm,
                  acc_out, den_out, sae_out, cnt_out,
                  src_v, dst_v, ae_v, ex_v, asrc_v, adst_v, rows_v, ones_v,
                  sem_a, sem_b,
                  acc_s, den_s, sae_s, cnt_s):
    cid = lax.axis_index("c")
    sid = lax.axis_index("s")
    wid = cid * NS + sid

    pltpu.sync_copy(src_hbm.at[wid], src_v)
    pltpu.sync_copy(dst_hbm.at[wid], dst_v)
    pltpu.sync_copy(ae_hbm.at[wid], ae_v)
    pltpu.sync_copy(asrc_hbm, asrc_v)
    pltpu.sync_copy(adst_hbm, adst_v)

    zeros16 = jnp.zeros((16,), jnp.float32)

    def _zrow(i, c):
        rows_v[0, i // 8, pl.ds((i % 8) * 16, 16)] = zeros16
        return c
    lax.fori_loop(0, K * 8, _zrow, 0)
    for i in range(K // 16):
        ones_v[pl.ds(i * 16, 16)] = zeros16 + 1.0

    base = sid * STRIPE
    for j in range(STRIPE // K):
        pltpu.sync_copy(rows_v.at[0], acc_s.at[pl.ds(base + j * K, K)])
        pltpu.sync_copy(rows_v.at[0, 0], den_s.at[pl.ds(base + j * K, K)])
        pltpu.sync_copy(rows_v.at[0, 0], sae_s.at[pl.ds(base + j * K, K)])
        pltpu.sync_copy(rows_v.at[0, 0], cnt_s.at[pl.ds(base + j * K, K)])
    plsc.subcore_barrier()

    def _chunk(g, c):
        for i in range(K // 16):
            s16 = src_v[g, pl.ds(i * 16, 16)]
            d16 = dst_v[g, pl.ds(i * 16, 16)]
            a16 = ae_v[g, pl.ds(i * 16, 16)]
            al = plsc.load_gather(asrc_v, [s16]) + plsc.load_gather(adst_v, [d16]) + a16
            al = jnp.where(al >= 0.0, al, 0.2 * al)
            ex_v[g, pl.ds(i * 16, 16)] = jnp.exp(al)

        pltpu.async_copy(h_hbm.at[src_v.at[g]], rows_v.at[0], sem_a).wait()

        g16 = jnp.full((16,), g, jnp.int32)

        def _scale(e, c2):
            exv = plsc.load_gather(ex_v, [g16, jnp.full((16,), e, jnp.int32)])
            for j in range(D // 16):
                rows_v[0, e, pl.ds(j * 16, 16)] = rows_v[0, e, pl.ds(j * 16, 16)] * exv
            return c2
        lax.fori_loop(0, K, _scale, 0)

        pltpu.sync_copy(rows_v.at[0], acc_s.at[dst_v.at[g]], add=True)
        pltpu.sync_copy(ex_v.at[g], den_s.at[dst_v.at[g]], add=True)
        pltpu.sync_copy(ae_v.at[g], sae_s.at[dst_v.at[g]], add=True)
        pltpu.sync_copy(ones_v, cnt_s.at[dst_v.at[g]], add=True)
        return c
    lax.fori_loop(0, NCH, _chunk, 0)

    plsc.subcore_barrier()

    pltpu.sync_copy(acc_s.at[pl.ds(base, STRIPE)], acc_out.at[cid, pl.ds(base, STRIPE)])
    pltpu.sync_copy(den_s.at[pl.ds(base, STRIPE)], den_out.at[cid, pl.ds(base, STRIPE)])
    pltpu.sync_copy(sae_s.at[pl.ds(base, STRIPE)], sae_out.at[cid, pl.ds(base, STRIPE)])
    pltpu.sync_copy(cnt_s.at[pl.ds(base, STRIPE)], cnt_out.at[cid, pl.ds(base, STRIPE)])


def _fin_body(acc_ref, den_ref, sae_ref, cnt_ref, h_ref, asrc_ref, adst_ref,
              b_ref, g_ref, bb_ref, out_ref):
    acc = acc_ref[0] + acc_ref[1]
    den = den_ref[0] + den_ref[1]
    sae = sae_ref[0] + sae_ref[1]
    cnt = cnt_ref[0] + cnt_ref[1]
    loop_ae = sae / jnp.maximum(cnt, 1.0)
    al = asrc_ref[...] + adst_ref[...] + loop_ae
    al = jnp.where(al >= 0.0, al, 0.2 * al)
    exl = jnp.exp(al)
    o = (acc + h_ref[...] * exl) / (den + exl + 1e-16) + b_ref[...]
    v = o[:N, :]
    mean = jnp.mean(v, axis=0, keepdims=True)
    var = jnp.mean((v - mean) ** 2, axis=0, keepdims=True)
    y = (v - mean) / jnp.sqrt(var + 1e-5) * g_ref[...] + bb_ref[...]
    out_ref[...] = jnp.maximum(y, 0.0)


def _finalize(acc2, den2, sae2, cnt2, h_pad, asrc_p, adst_p, b, g, bb):
    return pl.pallas_call(
        _fin_body,
        out_shape=jax.ShapeDtypeStruct((N, D), jnp.float32),
    )(acc2, den2.reshape(NC, N_PAD, 1), sae2.reshape(NC, N_PAD, 1),
      cnt2.reshape(NC, N_PAD, 1), h_pad, asrc_p.reshape(N_PAD, 1),
      adst_p.reshape(N_PAD, 1), b.reshape(1, D), g.reshape(1, D),
      bb.reshape(1, D))


def _layer(x, src3, dst3, ae3, W, att_s, att_d, b, bn_g, bn_b):
    h, asrc, adst = _project(x, W, att_s, att_d)
    h_pad = jnp.pad(h, ((0, N_PAD - N), (0, 0)))
    asrc_p = jnp.pad(asrc.reshape(N), (0, N_PAD - N))
    adst_p = jnp.pad(adst.reshape(N), (0, N_PAD - N))
    acc2, den2, sae2, cnt2 = _sc_edge_pass(h_pad, src3, dst3, ae3, asrc_p, adst_p)
    return _finalize(acc2, den2, sae2, cnt2, h_pad, asrc_p, adst_p, b, bn_g, bn_b)


def kernel(x, edge_index, edge_attr, W1, att_src1, att_dst1, We1, att_e1, b1,
           bn1_g, bn1_b, W2, att_src2, att_dst2, We2, att_e2, b2, bn2_g, bn2_b):
    src = edge_index[0].astype(jnp.int32)
    dst = edge_index[1].astype(jnp.int32)
    src3 = jnp.pad(src, (0, E_PAD - E)).reshape(NW, NCH, K)
    dst3 = jnp.pad(dst, (0, E_PAD - E), constant_values=DUMP).reshape(NW, NCH, K)

    ae1, ae2 = _edge_scores(edge_attr, We1, att_e1, We2, att_e2)
    ae1_3 = jnp.pad(ae1.reshape(E), (0, E_PAD - E), constant_values=_NEG).reshape(NW, NCH, K)
    ae2_3 = jnp.pad(ae2.reshape(E), (0, E_PAD - E), constant_values=_NEG).reshape(NW, NCH, K)

    y1 = _layer(x, src3, dst3, ae1_3, W1, att_src1, att_dst1, b1, bn1_g, bn1_b)
    y2 = _layer(y1, src3, dst3, ae2_3, W2, att_src2, att_dst2, b2, bn2_g, bn2_b)
    return y2


# speedup vs baseline: 17.1635x; 17.1635x over previous
import functools

import jax
import jax.numpy as jnp
from jax import lax
from jax.experimental import pallas as pl
from jax.experimental.pallas import tpu as pltpu
from jax.experimental.pallas import tpu_sc as plsc

N = 10000
E = 320000
D = 128
FE = 16

NC = 2
NS = 16
NW = NC * NS

K = 128
EPT = -(-E // (NW * K)) * K
NCH = EPT // K
E_PAD = EPT * NW

N_PAD = 10240
STRIPE = N_PAD // NS
DUMP = N

_NEG = -1e30


_EB = 8000


def _escore_body(ea_ref, we1_ref, ate1_ref, we2_ref, ate2_ref, o1_ref, o2_ref):
    i0 = lax.broadcasted_iota(jnp.int32, (128, 128), 0)
    i1 = lax.broadcasted_iota(jnp.int32, (128, 128), 1)
    mask = (i0 // 16 == i1).astype(jnp.float32)
    wv1 = jnp.dot(we1_ref[...], ate1_ref[...], preferred_element_type=jnp.float32)
    wv2 = jnp.dot(we2_ref[...], ate2_ref[...], preferred_element_type=jnp.float32)
    s1 = mask * jnp.concatenate([wv1] * 8, axis=0)
    s2 = mask * jnp.concatenate([wv2] * 8, axis=0)
    ea = ea_ref[...]
    o1_ref[...] = jnp.dot(ea, s1, preferred_element_type=jnp.float32)
    o2_ref[...] = jnp.dot(ea, s2, preferred_element_type=jnp.float32)


def _edge_scores(edge_attr, We1, att_e1, We2, att_e2):
    er = E // 8
    ea_r = edge_attr.reshape(er, 128)
    grid = er // _EB
    o1, o2 = pl.pallas_call(
        _escore_body,
        grid=(grid,),
        in_specs=[
            pl.BlockSpec((_EB, 128), lambda i: (i, 0)),
            pl.BlockSpec((FE, D), lambda i: (0, 0)),
            pl.BlockSpec((D, 1), lambda i: (0, 0)),
            pl.BlockSpec((FE, D), lambda i: (0, 0)),
            pl.BlockSpec((D, 1), lambda i: (0, 0)),
        ],
        out_specs=[
            pl.BlockSpec((_EB, 128), lambda i: (i, 0)),
            pl.BlockSpec((_EB, 128), lambda i: (i, 0)),
        ],
        out_shape=[
            jax.ShapeDtypeStruct((er, 128), jnp.float32),
            jax.ShapeDtypeStruct((er, 128), jnp.float32),
        ],
    )(ea_r, We1, att_e1.reshape(D, 1), We2, att_e2.reshape(D, 1))
    return o1[:, :8].reshape(E), o2[:, :8].reshape(E)


def _proj_body(x_ref, w_ref, ats_ref, atd_ref, h_ref, asrc_ref, adst_ref):
    h = jnp.dot(x_ref[...], w_ref[...], preferred_element_type=jnp.float32)
    h_ref[...] = h
    asrc_ref[...] = jnp.dot(h, ats_ref[...], preferred_element_type=jnp.float32)
    adst_ref[...] = jnp.dot(h, atd_ref[...], preferred_element_type=jnp.float32)


def _project(x, W, att_s, att_d):
    return pl.pallas_call(
        _proj_body,
        out_shape=[
            jax.ShapeDtypeStruct((N, D), jnp.float32),
            jax.ShapeDtypeStruct((N, 1), jnp.float32),
            jax.ShapeDtypeStruct((N, 1), jnp.float32),
        ],
    )(x, W, att_s.reshape(D, 1), att_d.reshape(D, 1))


_MESH = plsc.VectorSubcoreMesh(core_axis_name="c", subcore_axis_name="s")


@functools.partial(
    pl.kernel,
    out_type=[
        jax.ShapeDtypeStruct((NC, N_PAD, D), jnp.float32),
        jax.ShapeDtypeStruct((NC, N_PAD), jnp.float32),
        jax.ShapeDtypeStruct((NC, N_PAD), jnp.float32),
        jax.ShapeDtypeStruct((NC, N_PAD), jnp.float32),
    ],
    mesh=_MESH,
    compiler_params=pltpu.CompilerParams(needs_layout_passes=False),
    scratch_types=[
        pltpu.VMEM((K,), jnp.int32),
        pltpu.VMEM((K,), jnp.int32),
        pltpu.VMEM((K,), jnp.float32),
        pltpu.VMEM((K,), jnp.float32),
        pltpu.VMEM((N_PAD,), jnp.float32),
        pltpu.VMEM((N_PAD,), jnp.float32),
        pltpu.VMEM((K, D), jnp.float32),
        pltpu.VMEM((K,), jnp.float32),
        pltpu.SemaphoreType.DMA,
        pltpu.SemaphoreType.DMA,
        pltpu.VMEM_SHARED((N_PAD, D), jnp.float32),
        pltpu.VMEM_SHARED((N_PAD,), jnp.float32),
        pltpu.VMEM_SHARED((N_PAD,), jnp.float32),
        pltpu.VMEM_SHARED((N_PAD,), jnp.float32),
    ],
)
def _sc_edge_pass(h_hbm, src_hbm, dst_hbm, ae_hbm, asrc_hbm, adst_hbm,
                  acc_out, den_out, sae_out, cnt_out,
                  src_c, dst_c, ae_c, ex_c, asrc_v, adst_v, rows_v, ones_v,
                  sem_a, sem_b,
                  acc_s, den_s, sae_s, cnt_s):
    cid = lax.axis_index("c")
    sid = lax.axis_index("s")
    wid = cid * NS + sid

    pltpu.sync_copy(asrc_hbm, asrc_v)
    pltpu.sync_copy(adst_hbm, adst_v)

    zeros16 = jnp.zeros((16,), jnp.float32)

    def _zrow(i, c):
        rows_v[i // 8, pl.ds((i % 8) * 16, 16)] = zeros16
        return c
    lax.fori_loop(0, K * 8, _zrow, 0)
    for i in range(K // 16):
        ones_v[pl.ds(i * 16, 16)] = zeros16 + 1.0

    base = sid * STRIPE
    for j in range(STRIPE // K):
        pltpu.sync_copy(rows_v, acc_s.at[pl.ds(base + j * K, K)])
        pltpu.sync_copy(rows_v.at[0], den_s.at[pl.ds(base + j * K, K)])
        pltpu.sync_copy(rows_v.at[0], sae_s.at[pl.ds(base + j * K, K)])
        pltpu.sync_copy(rows_v.at[0], cnt_s.at[pl.ds(base + j * K, K)])
    plsc.subcore_barrier()

    def _chunk(g, c):
        pltpu.sync_copy(src_hbm.at[wid, g], src_c)
        pltpu.sync_copy(dst_hbm.at[wid, g], dst_c)
        pltpu.sync_copy(ae_hbm.at[wid, g], ae_c)

        hcp = pltpu.async_copy(h_hbm.at[src_c], rows_v, sem_a)

        for i in range(K // 16):
            s16 = src_c[pl.ds(i * 16, 16)]
            d16 = dst_c[pl.ds(i * 16, 16)]
            a16 = ae_c[pl.ds(i * 16, 16)]
            al = plsc.load_gather(asrc_v, [s16]) + plsc.load_gather(adst_v, [d16]) + a16
            al = jnp.where(al >= 0.0, al, 0.2 * al)
            ex_c[pl.ds(i * 16, 16)] = jnp.exp(al)
        hcp.wait()

        def _scale(e, c2):
            exv = plsc.load_gather(ex_c, [jnp.full((16,), e, jnp.int32)])
            for j in range(D // 16):
                rows_v[e, pl.ds(j * 16, 16)] = rows_v[e, pl.ds(j * 16, 16)] * exv
            return c2
        lax.fori_loop(0, K, _scale, 0)

        pltpu.sync_copy(rows_v, acc_s.at[dst_c], add=True)
        pltpu.sync_copy(ex_c, den_s.at[dst_c], add=True)
        pltpu.sync_copy(ae_c, sae_s.at[dst_c], add=True)
        pltpu.sync_copy(ones_v, cnt_s.at[dst_c], add=True)
        return c
    lax.fori_loop(0, NCH, _chunk, 0)

    plsc.subcore_barrier()

    pltpu.sync_copy(acc_s.at[pl.ds(base, STRIPE)], acc_out.at[cid, pl.ds(base, STRIPE)])
    pltpu.sync_copy(den_s.at[pl.ds(base, STRIPE)], den_out.at[cid, pl.ds(base, STRIPE)])
    pltpu.sync_copy(sae_s.at[pl.ds(base, STRIPE)], sae_out.at[cid, pl.ds(base, STRIPE)])
    pltpu.sync_copy(cnt_s.at[pl.ds(base, STRIPE)], cnt_out.at[cid, pl.ds(base, STRIPE)])


_NR = N_PAD // 128


def _scal_body(den_ref, sae_ref, cnt_ref, asrc_ref, adst_ref, exl_ref, invd_ref):
    den = den_ref[0] + den_ref[1]
    sae = sae_ref[0] + sae_ref[1]
    cnt = cnt_ref[0] + cnt_ref[1]
    loop_ae = sae / jnp.maximum(cnt, 1.0)
    al = asrc_ref[...] + adst_ref[...] + loop_ae
    al = jnp.where(al >= 0.0, al, 0.2 * al)
    exl = jnp.exp(al)
    exl_ref[...] = exl
    invd_ref[...] = 1.0 / (den + exl + 1e-16)


def _apply_body(acc_ref, h_ref, exl_ref, invd_ref, b_ref, g_ref, bb_ref, out_ref):
    o = (acc_ref[0] + acc_ref[1] + h_ref[...] * exl_ref[...]) * invd_ref[...] + b_ref[...]
    v = o[:N, :]
    mean = jnp.mean(v, axis=0, keepdims=True)
    var = jnp.mean((v - mean) ** 2, axis=0, keepdims=True)
    y = (v - mean) / jnp.sqrt(var + 1e-5) * g_ref[...] + bb_ref[...]
    out_ref[...] = jnp.maximum(y, 0.0)


def _finalize(acc2, den2, sae2, cnt2, h_pad, asrc_p, adst_p, b, g, bb):
    exl, invd = pl.pallas_call(
        _scal_body,
        out_shape=[
            jax.ShapeDtypeStruct((_NR, 128), jnp.float32),
            jax.ShapeDtypeStruct((_NR, 128), jnp.float32),
        ],
    )(den2.reshape(NC, _NR, 128), sae2.reshape(NC, _NR, 128),
      cnt2.reshape(NC, _NR, 128), asrc_p.reshape(_NR, 128),
      adst_p.reshape(_NR, 128))
    return pl.pallas_call(
        _apply_body,
        out_shape=jax.ShapeDtypeStruct((N, D), jnp.float32),
    )(acc2, h_pad, exl.reshape(N_PAD, 1), invd.reshape(N_PAD, 1),
      b.reshape(1, D), g.reshape(1, D), bb.reshape(1, D))


def _layer(x, src3, dst3, ae3, W, att_s, att_d, b, bn_g, bn_b):
    h, asrc, adst = _project(x, W, att_s, att_d)
    h_pad = jnp.pad(h, ((0, N_PAD - N), (0, 0)))
    asrc_p = jnp.pad(asrc.reshape(N), (0, N_PAD - N))
    adst_p = jnp.pad(adst.reshape(N), (0, N_PAD - N))
    acc2, den2, sae2, cnt2 = _sc_edge_pass(h_pad, src3, dst3, ae3, asrc_p, adst_p)
    return _finalize(acc2, den2, sae2, cnt2, h_pad, asrc_p, adst_p, b, bn_g, bn_b)


def kernel(x, edge_index, edge_attr, W1, att_src1, att_dst1, We1, att_e1, b1,
           bn1_g, bn1_b, W2, att_src2, att_dst2, We2, att_e2, b2, bn2_g, bn2_b):
    src = edge_index[0].astype(jnp.int32)
    dst = edge_index[1].astype(jnp.int32)
    src3 = jnp.pad(src, (0, E_PAD - E)).reshape(NW, NCH, K)
    dst3 = jnp.pad(dst, (0, E_PAD - E), constant_values=DUMP).reshape(NW, NCH, K)

    ae1, ae2 = _edge_scores(edge_attr, We1, att_e1, We2, att_e2)
    ae1_3 = jnp.pad(ae1.reshape(E), (0, E_PAD - E), constant_values=_NEG).reshape(NW, NCH, K)
    ae2_3 = jnp.pad(ae2.reshape(E), (0, E_PAD - E), constant_values=_NEG).reshape(NW, NCH, K)

    y1 = _layer(x, src3, dst3, ae1_3, W1, att_src1, att_dst1, b1, bn1_g, bn1_b)
    y2 = _layer(y1, src3, dst3, ae2_3, W2, att_src2, att_dst2, b2, bn2_g, bn2_b)
    return y2


# speedup vs baseline: 24.7215x; 1.4404x over previous
import functools

import jax
import jax.numpy as jnp
from jax import lax
from jax.experimental import pallas as pl
from jax.experimental.pallas import tpu as pltpu
from jax.experimental.pallas import tpu_sc as plsc

N = 10000
E = 320000
D = 128
FE = 16

NC = 2
NS = 16
NW = NC * NS

K = 64
NCH = 158
EPT = NCH * K
E_PAD = EPT * NW

N_PAD = 10240
STRIPE = N_PAD // NS
DUMP = N

_NEG = -1e30


_EB = 8000


def _escore_body(ea_ref, we1_ref, ate1_ref, we2_ref, ate2_ref, o1_ref, o2_ref):
    i0 = lax.broadcasted_iota(jnp.int32, (128, 128), 0)
    i1 = lax.broadcasted_iota(jnp.int32, (128, 128), 1)
    mask = (i0 // 16 == i1).astype(jnp.float32)
    wv1 = jnp.dot(we1_ref[...], ate1_ref[...], preferred_element_type=jnp.float32)
    wv2 = jnp.dot(we2_ref[...], ate2_ref[...], preferred_element_type=jnp.float32)
    s1 = mask * jnp.concatenate([wv1] * 8, axis=0)
    s2 = mask * jnp.concatenate([wv2] * 8, axis=0)
    ea = ea_ref[...]
    o1_ref[...] = jnp.dot(ea, s1, preferred_element_type=jnp.float32)
    o2_ref[...] = jnp.dot(ea, s2, preferred_element_type=jnp.float32)


def _edge_scores(edge_attr, We1, att_e1, We2, att_e2):
    er = E // 8
    ea_r = edge_attr.reshape(er, 128)
    grid = er // _EB
    o1, o2 = pl.pallas_call(
        _escore_body,
        grid=(grid,),
        in_specs=[
            pl.BlockSpec((_EB, 128), lambda i: (i, 0)),
            pl.BlockSpec((FE, D), lambda i: (0, 0)),
            pl.BlockSpec((D, 1), lambda i: (0, 0)),
            pl.BlockSpec((FE, D), lambda i: (0, 0)),
            pl.BlockSpec((D, 1), lambda i: (0, 0)),
        ],
        out_specs=[
            pl.BlockSpec((_EB, 128), lambda i: (i, 0)),
            pl.BlockSpec((_EB, 128), lambda i: (i, 0)),
        ],
        out_shape=[
            jax.ShapeDtypeStruct((er, 128), jnp.float32),
            jax.ShapeDtypeStruct((er, 128), jnp.float32),
        ],
    )(ea_r, We1, att_e1.reshape(D, 1), We2, att_e2.reshape(D, 1))
    return o1[:, :8].reshape(E), o2[:, :8].reshape(E)


def _proj_body(x_ref, w_ref, ats_ref, atd_ref, h_ref, asrc_ref, adst_ref):
    h = jnp.dot(x_ref[...], w_ref[...], preferred_element_type=jnp.float32)
    h_ref[...] = h
    asrc_ref[...] = jnp.dot(h, ats_ref[...], preferred_element_type=jnp.float32)
    adst_ref[...] = jnp.dot(h, atd_ref[...], preferred_element_type=jnp.float32)


def _project(x, W, att_s, att_d):
    return pl.pallas_call(
        _proj_body,
        out_shape=[
            jax.ShapeDtypeStruct((N, D), jnp.float32),
            jax.ShapeDtypeStruct((N, 1), jnp.float32),
            jax.ShapeDtypeStruct((N, 1), jnp.float32),
        ],
    )(x, W, att_s.reshape(D, 1), att_d.reshape(D, 1))


_MESH = plsc.VectorSubcoreMesh(core_axis_name="c", subcore_axis_name="s")

_TSRC = N
_TDST = N + 8


@functools.partial(
    pl.kernel,
    out_type=[
        jax.ShapeDtypeStruct((NC, N_PAD, D), jnp.float32),
        jax.ShapeDtypeStruct((NC, N_PAD), jnp.float32),
        jax.ShapeDtypeStruct((NC, N_PAD), jnp.float32),
        jax.ShapeDtypeStruct((NC, N_PAD), jnp.float32),
    ],
    mesh=_MESH,
    compiler_params=pltpu.CompilerParams(needs_layout_passes=False),
    scratch_types=[
        pltpu.VMEM((2, 2, K), jnp.int32),
        pltpu.VMEM((2, K), jnp.float32),
        pltpu.VMEM((2, K), jnp.float32),
        pltpu.VMEM((2, K), jnp.float32),
        pltpu.VMEM((2, K), jnp.int32),
        pltpu.VMEM((K,), jnp.float32),
        pltpu.VMEM((2, K, D), jnp.float32),
        pltpu.VMEM((_TSRC,), jnp.float32),
        pltpu.VMEM((_TDST,), jnp.float32),
        pltpu.SemaphoreType.DMA,
        pltpu.SemaphoreType.DMA,
        pltpu.SemaphoreType.DMA,
        pltpu.SemaphoreType.DMA,
        pltpu.SemaphoreType.DMA,
        pltpu.SemaphoreType.DMA,
        pltpu.VMEM_SHARED((N_PAD, D), jnp.float32),
        pltpu.VMEM_SHARED((N_PAD,), jnp.float32),
        pltpu.VMEM_SHARED((N_PAD,), jnp.float32),
        pltpu.VMEM_SHARED((N_PAD,), jnp.float32),
    ],
)
def _sc_edge_pass(h_hbm, ei_hbm, aein_hbm, asrc_hbm, adst_hbm, zscal_hbm,
                  acc_out, den_out, sae_out, cnt_out,
                  ei_v, aein_v, ex_v, ae_v, holdd_v, ones_v, rows_v,
                  asrc_v, adst_v,
                  sem_i0, sem_i1, sem_g0, sem_g1, sem_s0, sem_s1,
                  acc_s, den_s, sae_s, cnt_s):
    cid = lax.axis_index("c")
    sid = lax.axis_index("s")
    wid = cid * NS + sid
    sem_i = (sem_i0, sem_i1)
    sem_g = (sem_g0, sem_g1)
    sem_s = (sem_s0, sem_s1)
    base = sid * STRIPE

    def i_issue(g, b):
        pltpu.async_copy(ei_hbm.at[wid, g], ei_v.at[b], sem_i[b])
        pltpu.async_copy(aein_hbm.at[wid, g], aein_v.at[b], sem_i[b])

    def i_wait(b):
        pltpu.make_async_copy(ei_hbm.at[wid, 0], ei_v.at[b], sem_i[b]).wait()
        pltpu.make_async_copy(aein_hbm.at[wid, 0], aein_v.at[b], sem_i[b]).wait()

    def r_issue(b):
        pltpu.async_copy(h_hbm.at[ei_v.at[b, 0]], rows_v.at[b], sem_g[b])

    def r_wait(b):
        pltpu.make_async_copy(h_hbm.at[ei_v.at[b, 0]], rows_v.at[b], sem_g[b]).wait()

    def s_issue(b):
        pltpu.async_copy(rows_v.at[b], acc_s.at[holdd_v.at[b]], sem_s[b], add=True)
        pltpu.async_copy(ex_v.at[b], den_s.at[holdd_v.at[b]], sem_s[b], add=True)
        pltpu.async_copy(ae_v.at[b], sae_s.at[holdd_v.at[b]], sem_s[b], add=True)
        pltpu.async_copy(ones_v, cnt_s.at[holdd_v.at[b]], sem_s[b], add=True)

    def s_drain(b):
        pltpu.make_async_copy(rows_v.at[b], acc_s.at[holdd_v.at[b]], sem_s[b]).wait()
        pltpu.make_async_copy(ex_v.at[b], den_s.at[holdd_v.at[b]], sem_s[b]).wait()
        pltpu.make_async_copy(ae_v.at[b], sae_s.at[holdd_v.at[b]], sem_s[b]).wait()
        pltpu.make_async_copy(ones_v, cnt_s.at[holdd_v.at[b]], sem_s[b]).wait()

    def scalar_phase(b):
        for i in range(K // 16):
            sl = pl.ds(i * 16, 16)
            s16 = ei_v[b, 0, sl]
            d16 = ei_v[b, 1, sl]
            a16 = aein_v[b, sl]
            al = plsc.load_gather(asrc_v, [s16]) + plsc.load_gather(adst_v, [d16]) + a16
            al = jnp.where(al >= 0.0, al, 0.2 * al)
            ex_v[b, sl] = jnp.exp(al)
            ae_v[b, sl] = a16
            holdd_v[b, sl] = d16

    def scale(b):
        def _sc1(e, c):
            exv = plsc.load_gather(
                ex_v, [jnp.full((16,), b, jnp.int32), jnp.full((16,), e, jnp.int32)])
            for j in range(D // 16):
                rows_v[b, e, pl.ds(j * 16, 16)] = rows_v[b, e, pl.ds(j * 16, 16)] * exv
            return c
        lax.fori_loop(0, K, _sc1, 0, unroll=2)

    pltpu.sync_copy(asrc_hbm, asrc_v)
    pltpu.sync_copy(adst_hbm, adst_v)
    zeros16 = jnp.zeros((16,), jnp.float32)
    for i in range(K // 16):
        ones_v[pl.ds(i * 16, 16)] = zeros16 + 1.0

    def _zrow(i, c):
        rows_v[0, i // 8, pl.ds((i % 8) * 16, 16)] = zeros16
        return c
    lax.fori_loop(0, K * 8, _zrow, 0)
    for j in range(STRIPE // K):
        pltpu.sync_copy(rows_v.at[0], acc_s.at[pl.ds(base + j * K, K)])
    pltpu.sync_copy(zscal_hbm, den_s.at[pl.ds(base, STRIPE)])
    pltpu.sync_copy(zscal_hbm, sae_s.at[pl.ds(base, STRIPE)])
    pltpu.sync_copy(zscal_hbm, cnt_s.at[pl.ds(base, STRIPE)])
    plsc.subcore_barrier()

    def sub(g_new, b, live_new, drain):
        a = 1 - b
        @pl.when(live_new)
        def _():
            i_wait(b)

            @pl.when(drain)
            def _():
                s_drain(b)
            r_issue(b)
        scalar_phase(a)
        r_wait(a)

        @pl.when(g_new + 1 < NCH)
        def _():
            i_issue(g_new + 1, a)
        scale(a)
        s_issue(a)

    i_issue(0, 0)
    i_issue(1, 1)
    i_wait(0)
    r_issue(0)

    true_ = jnp.bool_(True)

    def _pair(gg, c):
        sub(2 * gg + 1, 1, true_, gg > 0)
        sub(2 * gg + 2, 0, 2 * gg + 2 < NCH, true_)
        return c
    lax.fori_loop(0, NCH // 2, _pair, 0)

    s_drain(0)
    s_drain(1)
    plsc.subcore_barrier()

    pltpu.sync_copy(acc_s.at[pl.ds(base, STRIPE)], acc_out.at[cid, pl.ds(base, STRIPE)])
    pltpu.sync_copy(den_s.at[pl.ds(base, STRIPE)], den_out.at[cid, pl.ds(base, STRIPE)])
    pltpu.sync_copy(sae_s.at[pl.ds(base, STRIPE)], sae_out.at[cid, pl.ds(base, STRIPE)])
    pltpu.sync_copy(cnt_s.at[pl.ds(base, STRIPE)], cnt_out.at[cid, pl.ds(base, STRIPE)])


_NR = N_PAD // 128


def _scal_body(den_ref, sae_ref, cnt_ref, asrc_ref, adst_ref, exl_ref, invd_ref):
    den = den_ref[0] + den_ref[1]
    sae = sae_ref[0] + sae_ref[1]
    cnt = cnt_ref[0] + cnt_ref[1]
    loop_ae = sae / jnp.maximum(cnt, 1.0)
    al = asrc_ref[...] + adst_ref[...] + loop_ae
    al = jnp.where(al >= 0.0, al, 0.2 * al)
    exl = jnp.exp(al)
    exl_ref[...] = exl
    invd_ref[...] = 1.0 / (den + exl + 1e-16)


def _apply_body(acc_ref, h_ref, exl_ref, invd_ref, b_ref, g_ref, bb_ref, out_ref):
    o = (acc_ref[0] + acc_ref[1] + h_ref[...] * exl_ref[...]) * invd_ref[...] + b_ref[...]
    v = o[:N, :]
    mean = jnp.mean(v, axis=0, keepdims=True)
    var = jnp.mean((v - mean) ** 2, axis=0, keepdims=True)
    y = (v - mean) / jnp.sqrt(var + 1e-5) * g_ref[...] + bb_ref[...]
    out_ref[...] = jnp.maximum(y, 0.0)


def _finalize(acc2, den2, sae2, cnt2, h_pad, asrc_p, adst_p, b, g, bb):
    exl, invd = pl.pallas_call(
        _scal_body,
        out_shape=[
            jax.ShapeDtypeStruct((_NR, 128), jnp.float32),
            jax.ShapeDtypeStruct((_NR, 128), jnp.float32),
        ],
    )(den2.reshape(NC, _NR, 128), sae2.reshape(NC, _NR, 128),
      cnt2.reshape(NC, _NR, 128), asrc_p.reshape(_NR, 128),
      adst_p.reshape(_NR, 128))
    return pl.pallas_call(
        _apply_body,
        out_shape=jax.ShapeDtypeStruct((N, D), jnp.float32),
    )(acc2, h_pad, exl.reshape(N_PAD, 1), invd.reshape(N_PAD, 1),
      b.reshape(1, D), g.reshape(1, D), bb.reshape(1, D))


def _layer(x, ei, aein, W, att_s, att_d, b, bn_g, bn_b, zscal):
    h, asrc, adst = _project(x, W, att_s, att_d)
    h_pad = jnp.pad(h, ((0, N_PAD - N), (0, 0)))
    asrc_p = jnp.pad(asrc.reshape(N), (0, N_PAD - N))
    adst_p = jnp.pad(adst.reshape(N), (0, N_PAD - N))
    adst_t = jnp.pad(adst.reshape(N), (0, _TDST - N))
    acc2, den2, sae2, cnt2 = _sc_edge_pass(
        h_pad, ei, aein, asrc.reshape(N), adst_t, zscal)
    return _finalize(acc2, den2, sae2, cnt2, h_pad, asrc_p, adst_p, b, bn_g, bn_b)


def kernel(x, edge_index, edge_attr, W1, att_src1, att_dst1, We1, att_e1, b1,
           bn1_g, bn1_b, W2, att_src2, att_dst2, We2, att_e2, b2, bn2_g, bn2_b):
    src = jnp.pad(edge_index[0].astype(jnp.int32), (0, E_PAD - E))
    dst = jnp.pad(edge_index[1].astype(jnp.int32), (0, E_PAD - E),
                  constant_values=DUMP)
    ei = jnp.stack([src, dst], axis=0).reshape(2, NW, NCH, K).transpose(1, 2, 0, 3)

    ae1, ae2 = _edge_scores(edge_attr, We1, att_e1, We2, att_e2)
    aein1 = jnp.pad(ae1, (0, E_PAD - E), constant_values=_NEG).reshape(NW, NCH, K)
    aein2 = jnp.pad(ae2, (0, E_PAD - E), constant_values=_NEG).reshape(NW, NCH, K)

    zscal = jnp.zeros((STRIPE,), jnp.float32)

    y1 = _layer(x, ei, aein1, W1, att_src1, att_dst1, b1, bn1_g, bn1_b, zscal)
    y2 = _layer(y1, ei, aein2, W2, att_src2, att_dst2, b2, bn2_g, bn2_b, zscal)
    return y2


# speedup vs baseline: 25.7791x; 1.0428x over previous
import functools

import jax
import jax.numpy as jnp
from jax import lax
from jax.experimental import pallas as pl
from jax.experimental.pallas import tpu as pltpu
from jax.experimental.pallas import tpu_sc as plsc

N = 10000
E = 320000
D = 128
FE = 16

NC = 2
NS = 16
NW = NC * NS

K = 64
NCH = 158
EPT = NCH * K
E_PAD = EPT * NW

N_PAD = 10240
STRIPE = N_PAD // NS
DUMP = N

_NEG = -1e30


_EB = 8000


def _escore_body(ea_ref, we1_ref, ate1_ref, we2_ref, ate2_ref, o1_ref, o2_ref):
    i0 = lax.broadcasted_iota(jnp.int32, (128, 128), 0)
    i1 = lax.broadcasted_iota(jnp.int32, (128, 128), 1)
    mask = (i0 // 16 == i1).astype(jnp.float32)
    wv1 = jnp.dot(we1_ref[...], ate1_ref[...], preferred_element_type=jnp.float32)
    wv2 = jnp.dot(we2_ref[...], ate2_ref[...], preferred_element_type=jnp.float32)
    s1 = mask * jnp.concatenate([wv1] * 8, axis=0)
    s2 = mask * jnp.concatenate([wv2] * 8, axis=0)
    ea = ea_ref[...]
    o1_ref[...] = jnp.dot(ea, s1, preferred_element_type=jnp.float32)
    o2_ref[...] = jnp.dot(ea, s2, preferred_element_type=jnp.float32)


def _edge_scores(edge_attr, We1, att_e1, We2, att_e2):
    er = E // 8
    ea_r = edge_attr.reshape(er, 128)
    grid = er // _EB
    o1, o2 = pl.pallas_call(
        _escore_body,
        grid=(grid,),
        in_specs=[
            pl.BlockSpec((_EB, 128), lambda i: (i, 0)),
            pl.BlockSpec((FE, D), lambda i: (0, 0)),
            pl.BlockSpec((D, 1), lambda i: (0, 0)),
            pl.BlockSpec((FE, D), lambda i: (0, 0)),
            pl.BlockSpec((D, 1), lambda i: (0, 0)),
        ],
        out_specs=[
            pl.BlockSpec((_EB, 128), lambda i: (i, 0)),
            pl.BlockSpec((_EB, 128), lambda i: (i, 0)),
        ],
        out_shape=[
            jax.ShapeDtypeStruct((er, 128), jnp.float32),
            jax.ShapeDtypeStruct((er, 128), jnp.float32),
        ],
    )(ea_r, We1, att_e1.reshape(D, 1), We2, att_e2.reshape(D, 1))
    return o1[:, :8].reshape(E), o2[:, :8].reshape(E)


def _proj_body(x_ref, w_ref, ats_ref, atd_ref, h_ref, asrc_ref, adst_ref):
    h = jnp.dot(x_ref[...], w_ref[...], preferred_element_type=jnp.float32)
    h_ref[...] = h
    asrc_ref[...] = jnp.dot(h, ats_ref[...], preferred_element_type=jnp.float32)
    adst_ref[...] = jnp.dot(h, atd_ref[...], preferred_element_type=jnp.float32)


def _project(x, W, att_s, att_d):
    return pl.pallas_call(
        _proj_body,
        out_shape=[
            jax.ShapeDtypeStruct((N, D), jnp.float32),
            jax.ShapeDtypeStruct((N, 1), jnp.float32),
            jax.ShapeDtypeStruct((N, 1), jnp.float32),
        ],
    )(x, W, att_s.reshape(D, 1), att_d.reshape(D, 1))


_MESH = plsc.VectorSubcoreMesh(core_axis_name="c", subcore_axis_name="s")

_TSRC = N
_TDST = N + 8


@functools.partial(
    pl.kernel,
    out_type=[
        jax.ShapeDtypeStruct((NC, N_PAD, D), jnp.float32),
        jax.ShapeDtypeStruct((NC, N_PAD), jnp.float32),
        jax.ShapeDtypeStruct((NC, N_PAD), jnp.float32),
        jax.ShapeDtypeStruct((NC, N_PAD), jnp.float32),
    ],
    mesh=_MESH,
    compiler_params=pltpu.CompilerParams(needs_layout_passes=False),
    scratch_types=[
        pltpu.VMEM((2, 2, K), jnp.int32),
        pltpu.VMEM((2, K), jnp.float32),
        pltpu.VMEM((2, K), jnp.float32),
        pltpu.VMEM((2, K), jnp.float32),
        pltpu.VMEM((2, K), jnp.int32),
        pltpu.VMEM((K,), jnp.float32),
        pltpu.VMEM((2, K, D), jnp.float32),
        pltpu.VMEM((_TSRC,), jnp.float32),
        pltpu.VMEM((_TDST,), jnp.float32),
        pltpu.SemaphoreType.DMA,
        pltpu.SemaphoreType.DMA,
        pltpu.SemaphoreType.DMA,
        pltpu.SemaphoreType.DMA,
        pltpu.SemaphoreType.DMA,
        pltpu.SemaphoreType.DMA,
        pltpu.VMEM_SHARED((N_PAD, D), jnp.float32),
        pltpu.VMEM_SHARED((N_PAD,), jnp.float32),
        pltpu.VMEM_SHARED((N_PAD,), jnp.float32),
        pltpu.VMEM_SHARED((N_PAD,), jnp.float32),
    ],
)
def _sc_edge_pass(h_hbm, ei_hbm, aein_hbm, asrc_hbm, adst_hbm, zscal_hbm,
                  acc_out, den_out, sae_out, cnt_out,
                  ei_v, aein_v, ex_v, ae_v, holdd_v, ones_v, rows_v,
                  asrc_v, adst_v,
                  sem_i0, sem_i1, sem_g0, sem_g1, sem_s0, sem_s1,
                  acc_s, den_s, sae_s, cnt_s):
    cid = lax.axis_index("c")
    sid = lax.axis_index("s")
    wid = cid * NS + sid
    sem_i = (sem_i0, sem_i1)
    sem_g = (sem_g0, sem_g1)
    sem_s = (sem_s0, sem_s1)
    base = sid * STRIPE

    def i_issue(g, b):
        pltpu.async_copy(ei_hbm.at[wid, g], ei_v.at[b], sem_i[b])
        pltpu.async_copy(aein_hbm.at[wid, g], aein_v.at[b], sem_i[b])

    def i_wait(b):
        pltpu.make_async_copy(ei_hbm.at[wid, 0], ei_v.at[b], sem_i[b]).wait()
        pltpu.make_async_copy(aein_hbm.at[wid, 0], aein_v.at[b], sem_i[b]).wait()

    def r_issue(b):
        pltpu.async_copy(h_hbm.at[ei_v.at[b, 0]], rows_v.at[b], sem_g[b])

    def r_wait(b):
        pltpu.make_async_copy(h_hbm.at[ei_v.at[b, 0]], rows_v.at[b], sem_g[b]).wait()

    def s_issue(b):
        pltpu.async_copy(rows_v.at[b], acc_s.at[holdd_v.at[b]], sem_s[b], add=True)
        pltpu.async_copy(ex_v.at[b], den_s.at[holdd_v.at[b]], sem_s[b], add=True)
        pltpu.async_copy(ae_v.at[b], sae_s.at[holdd_v.at[b]], sem_s[b], add=True)
        pltpu.async_copy(ones_v, cnt_s.at[holdd_v.at[b]], sem_s[b], add=True)

    def s_drain(b):
        pltpu.make_async_copy(rows_v.at[b], acc_s.at[holdd_v.at[b]], sem_s[b]).wait()
        pltpu.make_async_copy(ex_v.at[b], den_s.at[holdd_v.at[b]], sem_s[b]).wait()
        pltpu.make_async_copy(ae_v.at[b], sae_s.at[holdd_v.at[b]], sem_s[b]).wait()
        pltpu.make_async_copy(ones_v, cnt_s.at[holdd_v.at[b]], sem_s[b]).wait()

    def scalar_phase(b):
        for i in range(K // 16):
            sl = pl.ds(i * 16, 16)
            s16 = ei_v[b, 0, sl]
            d16 = ei_v[b, 1, sl]
            a16 = aein_v[b, sl]
            al = plsc.load_gather(asrc_v, [s16]) + plsc.load_gather(adst_v, [d16]) + a16
            al = jnp.where(al >= 0.0, al, 0.2 * al)
            ex_v[b, sl] = jnp.exp(al)
            ae_v[b, sl] = a16
            holdd_v[b, sl] = d16

    def scale(b):
        def _sc1(e, c):
            exv = plsc.load_gather(
                ex_v, [jnp.full((16,), b, jnp.int32), jnp.full((16,), e, jnp.int32)])
            for j in range(D // 16):
                rows_v[b, e, pl.ds(j * 16, 16)] = rows_v[b, e, pl.ds(j * 16, 16)] * exv
            return c
        lax.fori_loop(0, K, _sc1, 0, unroll=2)

    pltpu.sync_copy(asrc_hbm, asrc_v)
    pltpu.sync_copy(adst_hbm, adst_v)
    zeros16 = jnp.zeros((16,), jnp.float32)
    for i in range(K // 16):
        ones_v[pl.ds(i * 16, 16)] = zeros16 + 1.0

    def _zrow(i, c):
        rows_v[0, i // 8, pl.ds((i % 8) * 16, 16)] = zeros16
        return c
    lax.fori_loop(0, K * 8, _zrow, 0)
    for j in range(STRIPE // K):
        pltpu.sync_copy(rows_v.at[0], acc_s.at[pl.ds(base + j * K, K)])
    pltpu.sync_copy(zscal_hbm, den_s.at[pl.ds(base, STRIPE)])
    pltpu.sync_copy(zscal_hbm, sae_s.at[pl.ds(base, STRIPE)])
    pltpu.sync_copy(zscal_hbm, cnt_s.at[pl.ds(base, STRIPE)])
    plsc.subcore_barrier()

    def sub(g_new, b, live_new, drain):
        a = 1 - b
        @pl.when(live_new)
        def _():
            i_wait(b)

            @pl.when(drain)
            def _():
                s_drain(b)
            r_issue(b)
        scalar_phase(a)
        r_wait(a)

        @pl.when(g_new + 1 < NCH)
        def _():
            i_issue(g_new + 1, a)
        scale(a)
        s_issue(a)

    i_issue(0, 0)
    i_issue(1, 1)
    i_wait(0)
    r_issue(0)

    true_ = jnp.bool_(True)

    def _pair(gg, c):
        sub(2 * gg + 1, 1, true_, gg > 0)
        sub(2 * gg + 2, 0, 2 * gg + 2 < NCH, true_)
        return c
    lax.fori_loop(0, NCH // 2, _pair, 0)

    s_drain(0)
    s_drain(1)
    plsc.subcore_barrier()

    pltpu.sync_copy(acc_s.at[pl.ds(base, STRIPE)], acc_out.at[cid, pl.ds(base, STRIPE)])
    pltpu.sync_copy(den_s.at[pl.ds(base, STRIPE)], den_out.at[cid, pl.ds(base, STRIPE)])
    pltpu.sync_copy(sae_s.at[pl.ds(base, STRIPE)], sae_out.at[cid, pl.ds(base, STRIPE)])
    pltpu.sync_copy(cnt_s.at[pl.ds(base, STRIPE)], cnt_out.at[cid, pl.ds(base, STRIPE)])


_NR = N_PAD // 128


def _scal_body(den_ref, sae_ref, cnt_ref, asrc_ref, adst_ref, exl_ref, invd_ref):
    den = den_ref[0] + den_ref[1]
    sae = sae_ref[0] + sae_ref[1]
    cnt = cnt_ref[0] + cnt_ref[1]
    loop_ae = sae / jnp.maximum(cnt, 1.0)
    al = asrc_ref[...] + adst_ref[...] + loop_ae
    al = jnp.where(al >= 0.0, al, 0.2 * al)
    exl = jnp.exp(al)
    exl_ref[...] = exl
    invd_ref[...] = 1.0 / (den + exl + 1e-16)


def _bn_relu(o, g, bb):
    mean = jnp.mean(o, axis=0, keepdims=True)
    var = jnp.mean((o - mean) ** 2, axis=0, keepdims=True)
    return jnp.maximum((o - mean) / jnp.sqrt(var + 1e-5) * g + bb, 0.0)


def _apply_body(acc_ref, h_ref, exl_ref, invd_ref, b_ref, g_ref, bb_ref, out_ref):
    acc = acc_ref[0, :N, :] + acc_ref[1, :N, :]
    o = (acc + h_ref[...] * exl_ref[:N]) * invd_ref[:N] + b_ref[...]
    out_ref[...] = _bn_relu(o, g_ref[...], bb_ref[...])


def _applyproj_body(acc_ref, h_ref, exl_ref, invd_ref, b_ref, g_ref, bb_ref,
                    w_ref, ats_ref, atd_ref, h2_ref, asrc_ref, adst_ref):
    acc = acc_ref[0, :N, :] + acc_ref[1, :N, :]
    o = (acc + h_ref[...] * exl_ref[:N]) * invd_ref[:N] + b_ref[...]
    y = _bn_relu(o, g_ref[...], bb_ref[...])
    h2 = jnp.dot(y, w_ref[...], preferred_element_type=jnp.float32)
    h2_ref[...] = h2
    asrc_ref[...] = jnp.dot(h2, ats_ref[...], preferred_element_type=jnp.float32)
    adst_ref[...] = jnp.dot(h2, atd_ref[...], preferred_element_type=jnp.float32)


def _softmax_scalars(den2, sae2, cnt2, asrc_p, adst_p):
    return pl.pallas_call(
        _scal_body,
        out_shape=[
            jax.ShapeDtypeStruct((_NR, 128), jnp.float32),
            jax.ShapeDtypeStruct((_NR, 128), jnp.float32),
        ],
    )(den2.reshape(NC, _NR, 128), sae2.reshape(NC, _NR, 128),
      cnt2.reshape(NC, _NR, 128), asrc_p.reshape(_NR, 128),
      adst_p.reshape(_NR, 128))


def _finalize(acc2, den2, sae2, cnt2, h, asrc_p, adst_p, b, g, bb):
    exl, invd = _softmax_scalars(den2, sae2, cnt2, asrc_p, adst_p)
    return pl.pallas_call(
        _apply_body,
        out_shape=jax.ShapeDtypeStruct((N, D), jnp.float32),
    )(acc2, h, exl.reshape(N_PAD, 1), invd.reshape(N_PAD, 1),
      b.reshape(1, D), g.reshape(1, D), bb.reshape(1, D))


def _finalize_proj(acc2, den2, sae2, cnt2, h, asrc_p, adst_p, b, g, bb,
                   W2, ats2, atd2):
    exl, invd = _softmax_scalars(den2, sae2, cnt2, asrc_p, adst_p)
    return pl.pallas_call(
        _applyproj_body,
        out_shape=[
            jax.ShapeDtypeStruct((N, D), jnp.float32),
            jax.ShapeDtypeStruct((N, 1), jnp.float32),
            jax.ShapeDtypeStruct((N, 1), jnp.float32),
        ],
    )(acc2, h, exl.reshape(N_PAD, 1), invd.reshape(N_PAD, 1),
      b.reshape(1, D), g.reshape(1, D), bb.reshape(1, D),
      W2, ats2.reshape(D, 1), atd2.reshape(D, 1))


def _sc_call(h, asrc, adst, ei, aein, zscal):
    asrc_p = jnp.pad(asrc.reshape(N), (0, N_PAD - N))
    adst_p = jnp.pad(adst.reshape(N), (0, N_PAD - N))
    adst_t = jnp.pad(adst.reshape(N), (0, _TDST - N))
    acc2, den2, sae2, cnt2 = _sc_edge_pass(
        h, ei, aein, asrc.reshape(N), adst_t, zscal)
    return acc2, den2, sae2, cnt2, asrc_p, adst_p


def kernel(x, edge_index, edge_attr, W1, att_src1, att_dst1, We1, att_e1, b1,
           bn1_g, bn1_b, W2, att_src2, att_dst2, We2, att_e2, b2, bn2_g, bn2_b):
    src = jnp.pad(edge_index[0].astype(jnp.int32), (0, E_PAD - E))
    dst = jnp.pad(edge_index[1].astype(jnp.int32), (0, E_PAD - E),
                  constant_values=DUMP)
    ei = jnp.stack([src, dst], axis=0).reshape(2, NW, NCH, K).transpose(1, 2, 0, 3)

    ae1, ae2 = _edge_scores(edge_attr, We1, att_e1, We2, att_e2)
    aein1 = jnp.pad(ae1, (0, E_PAD - E), constant_values=_NEG).reshape(NW, NCH, K)
    aein2 = jnp.pad(ae2, (0, E_PAD - E), constant_values=_NEG).reshape(NW, NCH, K)

    zscal = jnp.zeros((STRIPE,), jnp.float32)

    h1, asrc1, adst1 = _project(x, W1, att_src1, att_dst1)
    acc2, den2, sae2, cnt2, asrc_p, adst_p = _sc_call(h1, asrc1, adst1, ei, aein1, zscal)
    h2, asrc2, adst2 = _finalize_proj(acc2, den2, sae2, cnt2, h1, asrc_p, adst_p,
                                      b1, bn1_g, bn1_b, W2, att_src2, att_dst2)
    acc2, den2, sae2, cnt2, asrc_p, adst_p = _sc_call(h2, asrc2, adst2, ei, aein2, zscal)
    return _finalize(acc2, den2, sae2, cnt2, h2, asrc_p, adst_p, b2, bn2_g, bn2_b)
